# initial kernel scaffold (unmeasured)
import jax
import jax.numpy as jnp
from jax import lax
from jax.experimental import pallas as pl
from jax.experimental.pallas import tpu as pltpu

N_DEV = 16
N_STEPS = 4
N_LAYERS = 3
N_EXCH = N_LAYERS * N_STEPS


def kernel(x, Win0, Wout0, Win1, Wout1, Win2, Wout2):
    b, d_loc = x.shape
    _, h_dim = Win0.shape

    def body(x_ref, win0_ref, wout0_ref, win1_ref, wout1_ref, win2_ref,
             wout2_ref, out_ref, accum_ref, recv_ref, send_sems, recv_sems):
        my = lax.axis_index("i")
        xb = x_ref[...].astype(jnp.bfloat16)
        wpairs = [(win0_ref, wout0_ref), (win1_ref, wout1_ref),
                  (win2_ref, wout2_ref)]
        for layer in range(N_LAYERS):
            win_ref, wout_ref = wpairs[layer]
            accum_ref[...] = jnp.dot(
                xb, win_ref[...].astype(jnp.bfloat16),
                preferred_element_type=jnp.float32)

            for k in range(N_STEPS):
                ex = layer * N_STEPS + k
                partner = my ^ (1 << k)
                rdma = pltpu.make_async_remote_copy(
                    src_ref=accum_ref,
                    dst_ref=recv_ref.at[ex],
                    send_sem=send_sems.at[ex],
                    recv_sem=recv_sems.at[ex],
                    device_id=(partner,),
                    device_id_type=pl.DeviceIdType.MESH,
                )
                rdma.start()
                rdma.wait()
                accum_ref[...] = accum_ref[...] + recv_ref[ex]

            hrelu = jnp.maximum(accum_ref[...], 0.0).astype(jnp.bfloat16)
            xnext = jnp.dot(hrelu, wout_ref[...].astype(jnp.bfloat16),
                            preferred_element_type=jnp.float32)
            if layer == N_LAYERS - 1:
                out_ref[...] = xnext
            else:
                xb = xnext.astype(jnp.bfloat16)

    return pl.pallas_call(
        body,
        out_shape=jax.ShapeDtypeStruct((b, d_loc), jnp.float32),
        in_specs=[pl.BlockSpec(memory_space=pltpu.VMEM)] * 7,
        out_specs=pl.BlockSpec(memory_space=pltpu.VMEM),
        scratch_shapes=[
            pltpu.VMEM((b, h_dim), jnp.float32),
            pltpu.VMEM((N_EXCH, b, h_dim), jnp.float32),
            pltpu.SemaphoreType.DMA((N_EXCH,)),
            pltpu.SemaphoreType.DMA((N_EXCH,)),
        ],
        compiler_params=pltpu.CompilerParams(collective_id=0),
    )(x, Win0, Wout0, Win1, Wout1, Win2, Wout2)


# baseline (device time: 86630 ns/iter reference)
import jax
import jax.numpy as jnp
from jax import lax
from jax.experimental import pallas as pl
from jax.experimental.pallas import tpu as pltpu

N_DEV = 16
N_STEPS = 4
N_LAYERS = 3
N_EXCH = N_LAYERS * N_STEPS


def kernel(x, Win0, Wout0, Win1, Wout1, Win2, Wout2):
    b, d_loc = x.shape
    _, h_dim = Win0.shape

    def body(x_ref, win0_ref, wout0_ref, win1_ref, wout1_ref, win2_ref,
             wout2_ref, out_ref, accum_ref, recv_ref, send_sems, recv_sems):
        my = lax.axis_index("i")
        xb = x_ref[...].astype(jnp.bfloat16)
        wpairs = [(win0_ref, wout0_ref), (win1_ref, wout1_ref),
                  (win2_ref, wout2_ref)]
        for layer in range(N_LAYERS):
            win_ref, wout_ref = wpairs[layer]
            accum_ref[...] = jnp.dot(
                xb, win_ref[...].astype(jnp.bfloat16),
                preferred_element_type=jnp.float32)

            for k in range(N_STEPS):
                ex = layer * N_STEPS + k
                partner = my ^ (1 << k)
                rdma = pltpu.make_async_remote_copy(
                    src_ref=accum_ref,
                    dst_ref=recv_ref.at[ex],
                    send_sem=send_sems.at[ex],
                    recv_sem=recv_sems.at[ex],
                    device_id=(partner,),
                    device_id_type=pl.DeviceIdType.MESH,
                )
                rdma.start()
                rdma.wait()
                accum_ref[...] = accum_ref[...] + recv_ref[ex]

            hrelu = jnp.maximum(accum_ref[...], 0.0).astype(jnp.bfloat16)
            xnext = jnp.dot(hrelu, wout_ref[...].astype(jnp.bfloat16),
                            preferred_element_type=jnp.float32)
            if layer == N_LAYERS - 1:
                out_ref[...] = xnext
            else:
                xb = xnext.astype(jnp.bfloat16)

    return pl.pallas_call(
        body,
        out_shape=jax.ShapeDtypeStruct((b, d_loc), jnp.float32),
        in_specs=[pl.BlockSpec(memory_space=pltpu.VMEM)] * 7,
        out_specs=pl.BlockSpec(memory_space=pltpu.VMEM),
        scratch_shapes=[
            pltpu.VMEM((b, h_dim), jnp.float32),
            pltpu.VMEM((N_EXCH, b, h_dim), jnp.float32),
            pltpu.SemaphoreType.DMA((N_EXCH,)),
            pltpu.SemaphoreType.DMA((N_EXCH,)),
        ],
    )(x, Win0, Wout0, Win1, Wout1, Win2, Wout2)


# device time: 64438 ns/iter; 1.3444x vs baseline; 1.3444x over previous
import jax
import jax.numpy as jnp
from jax import lax
from jax.experimental import pallas as pl
from jax.experimental.pallas import tpu as pltpu

N_DEV = 16
N_STEPS = 4
N_LAYERS = 3
N_EXCH = N_LAYERS * N_STEPS


def kernel(x, Win0, Wout0, Win1, Wout1, Win2, Wout2):
    b, d_loc = x.shape
    _, h_dim = Win0.shape

    def body(x_ref, win0_ref, wout0_ref, win1_ref, wout1_ref, win2_ref,
             wout2_ref, out_ref, accum_ref, send_ref, recv_ref,
             send_sems, recv_sems):
        my = lax.axis_index("i")
        xb = x_ref[...].astype(jnp.bfloat16)
        wpairs = [(win0_ref, wout0_ref), (win1_ref, wout1_ref),
                  (win2_ref, wout2_ref)]
        for layer in range(N_LAYERS):
            win_ref, wout_ref = wpairs[layer]
            accum_ref[...] = jnp.dot(
                xb, win_ref[...].astype(jnp.bfloat16),
                preferred_element_type=jnp.float32)

            for k in range(N_STEPS):
                ex = layer * N_STEPS + k
                partner = my ^ (1 << k)
                send_ref[ex] = accum_ref[...].astype(jnp.bfloat16)
                rdma = pltpu.make_async_remote_copy(
                    src_ref=send_ref.at[ex],
                    dst_ref=recv_ref.at[ex],
                    send_sem=send_sems.at[ex],
                    recv_sem=recv_sems.at[ex],
                    device_id=(partner,),
                    device_id_type=pl.DeviceIdType.MESH,
                )
                rdma.start()
                rdma.wait_recv()
                accum_ref[...] = accum_ref[...] + recv_ref[ex].astype(
                    jnp.float32)

            hrelu = jnp.maximum(accum_ref[...], 0.0).astype(jnp.bfloat16)
            xnext = jnp.dot(hrelu, wout_ref[...].astype(jnp.bfloat16),
                            preferred_element_type=jnp.float32)
            if layer == N_LAYERS - 1:
                out_ref[...] = xnext
            else:
                xb = xnext.astype(jnp.bfloat16)

        for ex in range(N_EXCH):
            drain = pltpu.make_async_remote_copy(
                src_ref=send_ref.at[ex],
                dst_ref=recv_ref.at[ex],
                send_sem=send_sems.at[ex],
                recv_sem=recv_sems.at[ex],
                device_id=(my,),
                device_id_type=pl.DeviceIdType.MESH,
            )
            drain.wait_send()

    return pl.pallas_call(
        body,
        out_shape=jax.ShapeDtypeStruct((b, d_loc), jnp.float32),
        in_specs=[pl.BlockSpec(memory_space=pltpu.VMEM)] * 7,
        out_specs=pl.BlockSpec(memory_space=pltpu.VMEM),
        scratch_shapes=[
            pltpu.VMEM((b, h_dim), jnp.float32),
            pltpu.VMEM((N_EXCH, b, h_dim), jnp.bfloat16),
            pltpu.VMEM((N_EXCH, b, h_dim), jnp.bfloat16),
            pltpu.SemaphoreType.DMA((N_EXCH,)),
            pltpu.SemaphoreType.DMA((N_EXCH,)),
        ],
    )(x, Win0, Wout0, Win1, Wout1, Win2, Wout2)
